# baseline (speedup 1.0000x reference)
import functools
import math

import jax
import jax.numpy as jnp
from jax import lax
from jax.experimental import pallas as pl
from jax.experimental.pallas import tpu as pltpu
from jax.experimental.pallas import tpu_sc as plsc

SEQ_LEN = 2048
BATCH = 64
DIM = 64
NC = 2
NS = 16
NW = NC * NS
N_HALF = 2
SEQ_H = SEQ_LEN // N_HALF
ROWS_H = SEQ_H * BATCH
ROWS = SEQ_LEN * BATCH
ROWS_W = ROWS_H // NW
SEQ_W = SEQ_H // NW
CHUNK_S = 2
CHUNK_R = CHUNK_S * BATCH
N_CHUNKS = SEQ_W // CHUNK_S
SCALE = math.sqrt(DIM)
LANES = 16
VPR = DIM // LANES
N_SLOTS = 4
LOOKAHEAD = 2
TC_BS = 64


def _sc_body(idx_hbm, wt_hbm, pe_hbm, out_hbm, idx_v, pe_v, bufs, gsems, osems):
    wid = lax.axis_index("s") * NC + lax.axis_index("c")
    base = wid * ROWS_W

    pltpu.sync_copy(idx_hbm.at[pl.ds(base, ROWS_W)], idx_v)
    pltpu.sync_copy(pe_hbm.at[pl.ds(wid * SEQ_W * DIM, SEQ_W * DIM)], pe_v)

    def start_gather(g):
        slot = g % N_SLOTS
        idx_slice = idx_v.at[pl.ds(g * CHUNK_R, CHUNK_R)]
        return pltpu.async_copy(wt_hbm.at[idx_slice], bufs.at[slot], gsems[slot])

    def start_out(g):
        slot = g % N_SLOTS
        return pltpu.async_copy(
            bufs.at[slot],
            out_hbm.at[pl.ds(base + g * CHUNK_R, CHUNK_R), pl.ds(0, DIM)],
            osems[slot],
        )

    gd = {}
    od = {}
    for g in range(LOOKAHEAD):
        gd[g] = start_gather(g)

    for g in range(N_CHUNKS):
        h = g + LOOKAHEAD
        if h < N_CHUNKS:
            prev = h - N_SLOTS
            if prev >= 0:
                od.pop(prev).wait()
            gd[h] = start_gather(h)

        gd.pop(g).wait()

        slot = g % N_SLOTS
        for sp in range(CHUNK_S):
            srow = g * CHUNK_S + sp
            pe_regs = [
                pe_v[pl.ds(srow * DIM + j * LANES, LANES)] for j in range(VPR)
            ]

            def row_body(r, c, pe_regs=pe_regs, sp=sp, slot=slot):
                k = sp * BATCH + r
                for j in range(VPR):
                    v = bufs[slot, k, pl.ds(j * LANES, LANES)]
                    bufs[slot, k, pl.ds(j * LANES, LANES)] = v * SCALE + pe_regs[j]
                return c

            lax.fori_loop(0, BATCH, row_body, 0, unroll=2)

        od[g] = start_out(g)

    for g in sorted(od):
        od.pop(g).wait()


@functools.cache
def _build_sc_half():
    mesh = plsc.VectorSubcoreMesh(
        core_axis_name="c", subcore_axis_name="s", num_cores=NC, num_subcores=NS
    )
    return pl.kernel(
        _sc_body,
        out_type=jax.ShapeDtypeStruct((ROWS_H, 128), jnp.float32),
        mesh=mesh,
        scratch_types=[
            pltpu.VMEM((ROWS_W,), jnp.int32),
            pltpu.VMEM((SEQ_W * DIM,), jnp.float32),
            pltpu.VMEM((N_SLOTS, CHUNK_R, DIM), jnp.float32),
            [pltpu.SemaphoreType.DMA] * N_SLOTS,
            [pltpu.SemaphoreType.DMA] * N_SLOTS,
        ],
        compiler_params=pltpu.CompilerParams(use_tc_tiling_on_sc=False),
    )


def _tc_slice_body(x_ref, o_ref):
    o_ref[...] = x_ref[:, :, :DIM]


def _tc_slice_carry_body(x_ref, acc_ref, o_ref):
    del acc_ref
    o_ref[...] = x_ref[:, :, :DIM]


@functools.cache
def _build_tc_first():
    return pl.pallas_call(
        _tc_slice_body,
        grid=(SEQ_H // TC_BS,),
        in_specs=[pl.BlockSpec((TC_BS, BATCH, 128), lambda i: (i, 0, 0))],
        out_specs=pl.BlockSpec((TC_BS, BATCH, DIM), lambda i: (i, 0, 0)),
        out_shape=jax.ShapeDtypeStruct((SEQ_LEN, BATCH, DIM), jnp.float32),
    )


@functools.cache
def _build_tc_second():
    nb = SEQ_H // TC_BS
    return pl.pallas_call(
        _tc_slice_carry_body,
        grid=(nb,),
        in_specs=[
            pl.BlockSpec((TC_BS, BATCH, 128), lambda i: (i, 0, 0)),
            pl.BlockSpec(memory_space=pl.ANY),
        ],
        out_specs=pl.BlockSpec((TC_BS, BATCH, DIM), lambda i, nb=nb: (i + nb, 0, 0)),
        out_shape=jax.ShapeDtypeStruct((SEQ_LEN, BATCH, DIM), jnp.float32),
        input_output_aliases={1: 0},
    )


def kernel(source, word_table, pe):
    idx = source.reshape(ROWS)
    pe_flat = pe[:SEQ_LEN, 0, :].reshape(SEQ_LEN * DIM)
    sc_half = _build_sc_half()
    h0 = sc_half(idx[:ROWS_H], word_table, pe_flat[: ROWS_H])
    h1 = sc_half(idx[ROWS_H:], word_table, pe_flat[ROWS_H:])
    a = h0.reshape(SEQ_H, BATCH, 128)
    b = h1.reshape(SEQ_H, BATCH, 128)
    o = _build_tc_first()(a)
    o = _build_tc_second()(b, o)
    return o


# speedup vs baseline: 1.0055x; 1.0055x over previous
import functools
import math

import jax
import jax.numpy as jnp
from jax import lax
from jax.experimental import pallas as pl
from jax.experimental.pallas import tpu as pltpu
from jax.experimental.pallas import tpu_sc as plsc

SEQ_LEN = 2048
BATCH = 64
DIM = 64
NC = 2
NS = 16
NW = NC * NS
N_HALF = 2
SEQ_H = SEQ_LEN // N_HALF
ROWS_H = SEQ_H * BATCH
ROWS = SEQ_LEN * BATCH
ROWS_W = ROWS_H // NW
SEQ_W = SEQ_H // NW
CHUNK_S = 2
CHUNK_R = CHUNK_S * BATCH
N_CHUNKS = SEQ_W // CHUNK_S
SCALE = math.sqrt(DIM)
LANES = 16
VPR = DIM // LANES
N_SLOTS = 4
LOOKAHEAD = 2
TC_BS = 64


def _sc_body(half, idx_hbm, wt_hbm, pe_hbm, out_hbm, idx_v, pe_v, bufs, gsems, osems):
    wid = lax.axis_index("s") * NC + lax.axis_index("c")
    base = wid * ROWS_W

    pltpu.sync_copy(idx_hbm.at[pl.ds(half * ROWS_H + base, ROWS_W)], idx_v)
    pltpu.sync_copy(
        pe_hbm.at[pl.ds(half * ROWS_H + wid * SEQ_W * DIM, SEQ_W * DIM)], pe_v
    )

    def start_gather(g):
        slot = g % N_SLOTS
        idx_slice = idx_v.at[pl.ds(g * CHUNK_R, CHUNK_R)]
        return pltpu.async_copy(wt_hbm.at[idx_slice], bufs.at[slot], gsems[slot])

    def start_out(g):
        slot = g % N_SLOTS
        return pltpu.async_copy(
            bufs.at[slot],
            out_hbm.at[pl.ds(base + g * CHUNK_R, CHUNK_R), pl.ds(0, DIM)],
            osems[slot],
        )

    gd = {}
    od = {}
    for g in range(LOOKAHEAD):
        gd[g] = start_gather(g)

    for g in range(N_CHUNKS):
        h = g + LOOKAHEAD
        if h < N_CHUNKS:
            prev = h - N_SLOTS
            if prev >= 0:
                od.pop(prev).wait()
            gd[h] = start_gather(h)

        gd.pop(g).wait()

        slot = g % N_SLOTS
        for sp in range(CHUNK_S):
            srow = g * CHUNK_S + sp
            pe_regs = [
                pe_v[pl.ds(srow * DIM + j * LANES, LANES)] for j in range(VPR)
            ]

            def row_body(r, c, pe_regs=pe_regs, sp=sp, slot=slot):
                k = sp * BATCH + r
                for j in range(VPR):
                    v = bufs[slot, k, pl.ds(j * LANES, LANES)]
                    bufs[slot, k, pl.ds(j * LANES, LANES)] = v * SCALE + pe_regs[j]
                return c

            lax.fori_loop(0, BATCH, row_body, 0, unroll=2)

        od[g] = start_out(g)

    for g in sorted(od):
        od.pop(g).wait()


@functools.cache
def _build_sc_half(half):
    mesh = plsc.VectorSubcoreMesh(
        core_axis_name="c", subcore_axis_name="s", num_cores=NC, num_subcores=NS
    )
    return pl.kernel(
        functools.partial(_sc_body, half),
        out_type=jax.ShapeDtypeStruct((ROWS_H, 128), jnp.float32),
        mesh=mesh,
        scratch_types=[
            pltpu.VMEM((ROWS_W,), jnp.int32),
            pltpu.VMEM((SEQ_W * DIM,), jnp.float32),
            pltpu.VMEM((N_SLOTS, CHUNK_R, DIM), jnp.float32),
            [pltpu.SemaphoreType.DMA] * N_SLOTS,
            [pltpu.SemaphoreType.DMA] * N_SLOTS,
        ],
        compiler_params=pltpu.CompilerParams(use_tc_tiling_on_sc=False),
    )


def _tc_slice_body(x_ref, o_ref):
    o_ref[...] = x_ref[:, :, :DIM]


def _tc_slice_carry_body(x_ref, acc_ref, o_ref):
    del acc_ref
    o_ref[...] = x_ref[:, :, :DIM]


@functools.cache
def _build_tc_first():
    return pl.pallas_call(
        _tc_slice_body,
        grid=(SEQ_H // TC_BS,),
        in_specs=[pl.BlockSpec((TC_BS, BATCH, 128), lambda i: (i, 0, 0))],
        out_specs=pl.BlockSpec((TC_BS, BATCH, DIM), lambda i: (i, 0, 0)),
        out_shape=jax.ShapeDtypeStruct((SEQ_LEN, BATCH, DIM), jnp.float32),
    )


@functools.cache
def _build_tc_second():
    nb = SEQ_H // TC_BS
    return pl.pallas_call(
        _tc_slice_carry_body,
        grid=(nb,),
        in_specs=[
            pl.BlockSpec((TC_BS, BATCH, 128), lambda i: (i, 0, 0)),
            pl.BlockSpec(memory_space=pl.ANY),
        ],
        out_specs=pl.BlockSpec((TC_BS, BATCH, DIM), lambda i, nb=nb: (i + nb, 0, 0)),
        out_shape=jax.ShapeDtypeStruct((SEQ_LEN, BATCH, DIM), jnp.float32),
        input_output_aliases={1: 0},
    )


def kernel(source, word_table, pe):
    idx = source.reshape(ROWS)
    pe_flat = pe[:SEQ_LEN, 0, :].reshape(SEQ_LEN * DIM)
    h0 = _build_sc_half(0)(idx, word_table, pe_flat)
    h1 = _build_sc_half(1)(idx, word_table, pe_flat)
    a = h0.reshape(SEQ_H, BATCH, 128)
    b = h1.reshape(SEQ_H, BATCH, 128)
    o = _build_tc_first()(a)
    o = _build_tc_second()(b, o)
    return o
